# Initial kernel scaffold; baseline (speedup 1.0000x reference)
import jax
import jax.numpy as jnp
from jax.experimental import pallas as pl


def kernel(hidden_states, gate_w, gate_proj, up_proj, down_proj, sh_gate, sh_up, sh_down):
    raise NotImplementedError("write your pallas kernel here")


# speedup vs baseline: 1.1583x; 1.1583x over previous
import functools

import jax
import jax.numpy as jnp
from jax import lax
from jax.experimental import pallas as pl
from jax.experimental.pallas import tpu as pltpu

K = 2
BM = 128
TB = 128


def _gate_shared_body(x_ref, gw_ref, sg_ref, su_ref, sd_ref,
                      ysh_ref, w_ref, i_ref):
    xb = x_ref[...]
    logits = lax.dot_general(xb, gw_ref[...], (((1,), (1,)), ((), ())),
                             preferred_element_type=jnp.float32)
    m = jnp.max(logits, axis=1, keepdims=True)
    ex = jnp.exp(logits - m)
    s = ex / jnp.sum(ex, axis=1, keepdims=True)
    n_e = s.shape[1]
    iota = lax.broadcasted_iota(jnp.int32, s.shape, 1)
    m1 = jnp.max(s, axis=1, keepdims=True)
    i1 = jnp.min(jnp.where(s == m1, iota, n_e), axis=1, keepdims=True)
    s2 = jnp.where(iota == i1, -1.0, s)
    m2 = jnp.max(s2, axis=1, keepdims=True)
    i2 = jnp.min(jnp.where(s2 == m2, iota, n_e), axis=1, keepdims=True)
    w_ref[...] = jnp.concatenate([m1, m2], axis=1)
    i_ref[...] = jnp.concatenate([i1, i2], axis=1).astype(jnp.int32)
    xb16 = xb.astype(jnp.bfloat16)
    a = jnp.dot(xb16, sg_ref[...], preferred_element_type=jnp.float32)
    b = jnp.dot(xb16, su_ref[...], preferred_element_type=jnp.float32)
    h = (jax.nn.silu(a) * b).astype(jnp.bfloat16)
    ysh_ref[...] = jnp.dot(h, sd_ref[...], preferred_element_type=jnp.float32)


def _expert_body(plan_ref, xs_ref, g_ref, u_ref, d_ref, out_ref):
    xb = xs_ref[...].astype(jnp.bfloat16)
    a = jnp.dot(xb, g_ref[0], preferred_element_type=jnp.float32)
    b = jnp.dot(xb, u_ref[0], preferred_element_type=jnp.float32)
    h = (jax.nn.silu(a) * b).astype(jnp.bfloat16)
    out_ref[...] = jnp.dot(h, d_ref[0], preferred_element_type=jnp.float32)


def _combine_body(g_ref, w_ref, ysh_ref, y_ref):
    g = g_ref[...]
    d = g.shape[-1]
    gg = g.reshape(TB, K, d)
    w = w_ref[...]
    y_ref[...] = (gg * w[:, :, None]).sum(axis=1) + ysh_ref[...]


def kernel(hidden_states, gate_w, gate_proj, up_proj, down_proj,
           sh_gate, sh_up, sh_down):
    orig_shape = hidden_states.shape
    n_e, d_model, d_ff = gate_proj.shape
    t = hidden_states.shape[0] * hidden_states.shape[1]
    p_tot = t * K
    nb = p_tot // BM + n_e - 1
    x = hidden_states.reshape(t, d_model)

    gp16 = gate_proj.astype(jnp.bfloat16)
    up16 = up_proj.astype(jnp.bfloat16)
    dp16 = down_proj.astype(jnp.bfloat16)
    sg16 = sh_gate.astype(jnp.bfloat16)
    su16 = sh_up.astype(jnp.bfloat16)
    sd16 = sh_down.astype(jnp.bfloat16)

    ysh, topk_w, topk_i = pl.pallas_call(
        _gate_shared_body,
        grid=(t // TB,),
        in_specs=[
            pl.BlockSpec((TB, d_model), lambda b: (b, 0)),
            pl.BlockSpec(gate_w.shape, lambda b: (0, 0)),
            pl.BlockSpec(sg16.shape, lambda b: (0, 0)),
            pl.BlockSpec(su16.shape, lambda b: (0, 0)),
            pl.BlockSpec(sd16.shape, lambda b: (0, 0)),
        ],
        out_specs=[
            pl.BlockSpec((TB, d_model), lambda b: (b, 0)),
            pl.BlockSpec((TB, K), lambda b: (b, 0)),
            pl.BlockSpec((TB, K), lambda b: (b, 0)),
        ],
        out_shape=[
            jax.ShapeDtypeStruct((t, d_model), jnp.float32),
            jax.ShapeDtypeStruct((t, K), jnp.float32),
            jax.ShapeDtypeStruct((t, K), jnp.int32),
        ],
    )(x, gate_w, sg16, su16, sd16)

    e_flat = topk_i.reshape(-1)
    oh = (e_flat[:, None] == jnp.arange(n_e, dtype=jnp.int32)).astype(jnp.int32)
    counts = oh.sum(0)
    rank = ((jnp.cumsum(oh, axis=0) - oh) * oh).sum(1)
    blocks_e = (counts + BM - 1) // BM
    seg_start = (jnp.cumsum(blocks_e) - blocks_e) * BM
    pad_pos = seg_start[e_flat] + rank
    cum_blocks = jnp.cumsum(blocks_e)
    plan = jnp.minimum(
        jnp.searchsorted(cum_blocks, jnp.arange(nb), side='right'),
        n_e - 1).astype(jnp.int32)

    xs = jnp.zeros((nb * BM, d_model), jnp.float32).at[pad_pos].set(
        jnp.repeat(x, K, axis=0))

    out_pairs = pl.pallas_call(
        _expert_body,
        grid_spec=pltpu.PrefetchScalarGridSpec(
            num_scalar_prefetch=1,
            grid=(nb,),
            in_specs=[
                pl.BlockSpec((BM, d_model), lambda b, pr: (b, 0)),
                pl.BlockSpec((1, d_model, d_ff), lambda b, pr: (pr[b], 0, 0)),
                pl.BlockSpec((1, d_model, d_ff), lambda b, pr: (pr[b], 0, 0)),
                pl.BlockSpec((1, d_ff, d_model), lambda b, pr: (pr[b], 0, 0)),
            ],
            out_specs=pl.BlockSpec((BM, d_model), lambda b, pr: (b, 0)),
        ),
        out_shape=jax.ShapeDtypeStruct((nb * BM, d_model), jnp.float32),
    )(plan, xs, gp16, up16, dp16)

    gath = out_pairs[pad_pos]

    y = pl.pallas_call(
        _combine_body,
        grid=(t // TB,),
        in_specs=[
            pl.BlockSpec((K * TB, d_model), lambda b: (b, 0)),
            pl.BlockSpec((TB, K), lambda b: (b, 0)),
            pl.BlockSpec((TB, d_model), lambda b: (b, 0)),
        ],
        out_specs=pl.BlockSpec((TB, d_model), lambda b: (b, 0)),
        out_shape=jax.ShapeDtypeStruct((t, d_model), jnp.float32),
    )(gath, topk_w, ysh)

    return y.reshape(orig_shape)


# speedup vs baseline: 1.8817x; 1.6245x over previous
import functools

import jax
import jax.numpy as jnp
from jax import lax
from jax.experimental import pallas as pl
from jax.experimental.pallas import tpu as pltpu
from jax.experimental.pallas import tpu_sc as plsc

K = 2
BM = 64
TB = 128
NBLK = 128
NW = 32


def _gate_body(x_ref, gw_ref, w_ref, i_ref):
    xb = x_ref[...]
    logits = lax.dot_general(xb, gw_ref[...], (((1,), (1,)), ((), ())),
                             preferred_element_type=jnp.float32)
    m = jnp.max(logits, axis=1, keepdims=True)
    ex = jnp.exp(logits - m)
    s = ex / jnp.sum(ex, axis=1, keepdims=True)
    n_e = s.shape[1]
    iota = lax.broadcasted_iota(jnp.int32, s.shape, 1)
    m1 = jnp.max(s, axis=1, keepdims=True)
    i1 = jnp.min(jnp.where(s == m1, iota, n_e), axis=1, keepdims=True)
    s2 = jnp.where(iota == i1, -1.0, s)
    m2 = jnp.max(s2, axis=1, keepdims=True)
    i2 = jnp.min(jnp.where(s2 == m2, iota, n_e), axis=1, keepdims=True)
    w_ref[...] = jnp.concatenate([m1, m2], axis=1)
    i_ref[...] = jnp.concatenate([i1, i2], axis=1).astype(jnp.int32)


def _shared_body(x_ref, sg_ref, su_ref, sd_ref, ysh_ref):
    xb16 = x_ref[...].astype(jnp.bfloat16)
    a = jnp.dot(xb16, sg_ref[...], preferred_element_type=jnp.float32)
    b = jnp.dot(xb16, su_ref[...], preferred_element_type=jnp.float32)
    h = (jax.nn.silu(a) * b).astype(jnp.bfloat16)
    ysh_ref[...] = jnp.dot(h, sd_ref[...], preferred_element_type=jnp.float32)


def _route_body(ti_ref, p0_ref, p1_ref, plan_ref):
    g = pl.program_id(0)
    e = ti_ref[...]
    t = e.shape[0]
    rb = p0_ref.shape[0]
    n_e = 64
    iota_e = lax.broadcasted_iota(jnp.int32, (t, n_e), 1)
    oh0 = (e[:, 0:1] == iota_e)
    oh1 = (e[:, 1:2] == iota_e)
    both = oh0.astype(jnp.bfloat16) + oh1.astype(jnp.bfloat16)
    counts = jnp.sum(both.astype(jnp.float32), axis=0, keepdims=True)
    blocks = jnp.ceil(counts * (1.0 / BM))
    rr = lax.broadcasted_iota(jnp.int32, (n_e, n_e), 0)
    cc = lax.broadcasted_iota(jnp.int32, (n_e, n_e), 1)
    w_lt = (rr < cc).astype(jnp.float32)
    startblk = jnp.dot(blocks, w_lt, preferred_element_type=jnp.float32)
    seg_start = startblk * BM
    row_i = lax.broadcasted_iota(jnp.int32, (rb, t), 0) + g * rb
    col_i = lax.broadcasted_iota(jnp.int32, (rb, t), 1)
    mask = (col_i < row_i).astype(jnp.bfloat16)
    below = jnp.dot(mask, both, preferred_element_type=jnp.float32)
    iota_eb = lax.broadcasted_iota(jnp.int32, (rb, n_e), 1)
    eb = ti_ref[pl.ds(g * rb, rb), :]
    oh0b = (eb[:, 0:1] == iota_eb).astype(jnp.float32)
    oh1b = (eb[:, 1:2] == iota_eb).astype(jnp.float32)
    base = below + seg_start
    p0 = jnp.sum(oh0b * base, axis=1, keepdims=True)
    p1 = jnp.sum(oh1b * (base + oh0b), axis=1, keepdims=True)
    p0_ref[...] = p0.astype(jnp.int32)
    p1_ref[...] = p1.astype(jnp.int32)
    iota_b = lax.broadcasted_iota(jnp.int32, (NBLK, n_e), 0)
    plan = jnp.sum((startblk.astype(jnp.int32) <= iota_b).astype(jnp.int32),
                   axis=1, keepdims=True) - 1
    plan_ref[...] = plan


def _expert_body(plan_ref, xs_ref, g_ref, u_ref, d_ref, out_ref):
    xb = xs_ref[...].astype(jnp.bfloat16)
    g16 = g_ref[0].astype(jnp.bfloat16)
    u16 = u_ref[0].astype(jnp.bfloat16)
    d16 = d_ref[0].astype(jnp.bfloat16)
    a = jnp.dot(xb, g16, preferred_element_type=jnp.float32)
    b = jnp.dot(xb, u16, preferred_element_type=jnp.float32)
    h = (jax.nn.silu(a) * b).astype(jnp.bfloat16)
    out_ref[...] = jnp.dot(h, d16, preferred_element_type=jnp.float32)


def _combine_body(g0_ref, g1_ref, w_ref, ysh_ref, y_ref):
    w = w_ref[...]
    y_ref[...] = (g0_ref[...] * w[:, 0:1] + g1_ref[...] * w[:, 1:2]
                  + ysh_ref[...])


def _make_dispatch(t, d, r, tw):
    mesh = plsc.VectorSubcoreMesh(core_axis_name="c", subcore_axis_name="s")

    @functools.partial(
        pl.kernel, mesh=mesh,
        out_type=jax.ShapeDtypeStruct((r, d), jnp.float32),
        scratch_types=[
            pltpu.VMEM((tw,), jnp.int32),
            pltpu.VMEM((tw,), jnp.int32),
            pltpu.VMEM((tw, d), jnp.float32),
            pltpu.SemaphoreType.DMA,
        ],
    )
    def dispatch(x_hbm, pos0_hbm, pos1_hbm, out_hbm, i0_v, i1_v, rows_v, sem):
        wid = lax.axis_index("s") * 2 + lax.axis_index("c")
        base = wid * tw
        pltpu.sync_copy(pos0_hbm.at[pl.ds(base, tw)], i0_v)
        pltpu.sync_copy(pos1_hbm.at[pl.ds(base, tw)], i1_v)
        pltpu.sync_copy(x_hbm.at[pl.ds(base, tw)], rows_v)
        a = pltpu.async_copy(rows_v, out_hbm.at[i0_v], sem)
        b = pltpu.async_copy(rows_v, out_hbm.at[i1_v], sem)
        a.wait()
        b.wait()

    return dispatch


def _make_gather(t, d, r, tw):
    mesh = plsc.VectorSubcoreMesh(core_axis_name="c", subcore_axis_name="s")

    @functools.partial(
        pl.kernel, mesh=mesh,
        out_type=[jax.ShapeDtypeStruct((t, d), jnp.float32),
                  jax.ShapeDtypeStruct((t, d), jnp.float32)],
        scratch_types=[
            pltpu.VMEM((tw,), jnp.int32),
            pltpu.VMEM((tw, d), jnp.float32),
            pltpu.SemaphoreType.DMA,
        ],
    )
    def gather(outp_hbm, pos0_hbm, pos1_hbm, g0_hbm, g1_hbm,
               i_v, r_v, sem):
        wid = lax.axis_index("s") * 2 + lax.axis_index("c")
        base = wid * tw
        pltpu.sync_copy(pos0_hbm.at[pl.ds(base, tw)], i_v)
        pltpu.async_copy(outp_hbm.at[i_v], r_v, sem).wait()
        pltpu.sync_copy(r_v, g0_hbm.at[pl.ds(base, tw)])
        pltpu.sync_copy(pos1_hbm.at[pl.ds(base, tw)], i_v)
        pltpu.async_copy(outp_hbm.at[i_v], r_v, sem).wait()
        pltpu.sync_copy(r_v, g1_hbm.at[pl.ds(base, tw)])

    return gather


def kernel(hidden_states, gate_w, gate_proj, up_proj, down_proj,
           sh_gate, sh_up, sh_down):
    orig_shape = hidden_states.shape
    n_e, d_model, d_ff = gate_proj.shape
    t = hidden_states.shape[0] * hidden_states.shape[1]
    r = NBLK * BM
    tw = t // NW
    x = hidden_states.reshape(t, d_model)

    sg16 = sh_gate.astype(jnp.bfloat16)
    su16 = sh_up.astype(jnp.bfloat16)
    sd16 = sh_down.astype(jnp.bfloat16)

    topk_w, topk_i = pl.pallas_call(
        _gate_body,
        grid=(t // TB,),
        in_specs=[
            pl.BlockSpec((TB, d_model), lambda b: (b, 0)),
            pl.BlockSpec(gate_w.shape, lambda b: (0, 0)),
        ],
        out_specs=[
            pl.BlockSpec((TB, K), lambda b: (b, 0)),
            pl.BlockSpec((TB, K), lambda b: (b, 0)),
        ],
        out_shape=[
            jax.ShapeDtypeStruct((t, K), jnp.float32),
            jax.ShapeDtypeStruct((t, K), jnp.int32),
        ],
    )(x, gate_w)

    rb = t // 16
    pos0, pos1, plan = pl.pallas_call(
        _route_body,
        grid=(16,),
        in_specs=[pl.BlockSpec((t, K), lambda g: (0, 0))],
        out_specs=[
            pl.BlockSpec((rb, 1), lambda g: (g, 0)),
            pl.BlockSpec((rb, 1), lambda g: (g, 0)),
            pl.BlockSpec((NBLK, 1), lambda g: (0, 0)),
        ],
        out_shape=[
            jax.ShapeDtypeStruct((t, 1), jnp.int32),
            jax.ShapeDtypeStruct((t, 1), jnp.int32),
            jax.ShapeDtypeStruct((NBLK, 1), jnp.int32),
        ],
    )(topk_i)
    pos0 = pos0.reshape(t)
    pos1 = pos1.reshape(t)
    plan = plan.reshape(NBLK)

    xs = _make_dispatch(t, d_model, r, tw)(x, pos0, pos1)

    ysh = pl.pallas_call(
        _shared_body,
        grid=(t // TB,),
        in_specs=[
            pl.BlockSpec((TB, d_model), lambda b: (b, 0)),
            pl.BlockSpec(sg16.shape, lambda b: (0, 0)),
            pl.BlockSpec(su16.shape, lambda b: (0, 0)),
            pl.BlockSpec(sd16.shape, lambda b: (0, 0)),
        ],
        out_specs=pl.BlockSpec((TB, d_model), lambda b: (b, 0)),
        out_shape=jax.ShapeDtypeStruct((t, d_model), jnp.float32),
    )(x, sg16, su16, sd16)

    out_pairs = pl.pallas_call(
        _expert_body,
        grid_spec=pltpu.PrefetchScalarGridSpec(
            num_scalar_prefetch=1,
            grid=(NBLK,),
            in_specs=[
                pl.BlockSpec((BM, d_model), lambda b, pr: (b, 0)),
                pl.BlockSpec((1, d_model, d_ff), lambda b, pr: (pr[b], 0, 0)),
                pl.BlockSpec((1, d_model, d_ff), lambda b, pr: (pr[b], 0, 0)),
                pl.BlockSpec((1, d_ff, d_model), lambda b, pr: (pr[b], 0, 0)),
            ],
            out_specs=pl.BlockSpec((BM, d_model), lambda b, pr: (b, 0)),
        ),
        out_shape=jax.ShapeDtypeStruct((r, d_model), jnp.float32),
    )(plan, xs, gate_proj, up_proj, down_proj)

    g0, g1 = _make_gather(t, d_model, r, tw)(out_pairs, pos0, pos1)

    y = pl.pallas_call(
        _combine_body,
        grid=(t // TB,),
        in_specs=[
            pl.BlockSpec((TB, d_model), lambda b: (b, 0)),
            pl.BlockSpec((TB, d_model), lambda b: (b, 0)),
            pl.BlockSpec((TB, K), lambda b: (b, 0)),
            pl.BlockSpec((TB, d_model), lambda b: (b, 0)),
        ],
        out_specs=pl.BlockSpec((TB, d_model), lambda b: (b, 0)),
        out_shape=jax.ShapeDtypeStruct((t, d_model), jnp.float32),
    )(g0, g1, topk_w, ysh)

    return y.reshape(orig_shape)
